# baseline (speedup 1.0000x reference)
import jax
import jax.numpy as jnp
from jax.experimental import pallas as pl
from jax.experimental.pallas import tpu as pltpu

B, T, C, NH, KS = 4, 2048, 768, 12, 16
HD = C // NH
G = NH * KS
TS = 1024


def _dot(a, b, dims):
    return jax.lax.dot_general(a, b, (dims, ((), ())),
                               preferred_element_type=jnp.float32)


def _fused_kernel(x_ref, x16_ref, wqkv_ref, wp_ref, kms_ref, vm_ref, m_ref,
                  g_ref, gt_ref, o_ref, kbd_s, vbd_s, wqb_s, wpb_s):
    b = pl.program_id(0)
    i = pl.program_id(1)
    bf16 = jnp.bfloat16

    @pl.when((b == 0) & (i == 0))
    def _init():
        kT = _dot(wqkv_ref[C:2 * C, :], x16_ref[:], ((1,), (1,)))
        v = _dot(x16_ref[:], wqkv_ref[2 * C:, :], ((1,), (1,)))
        for bb in range(B):
            kb = kT[:, bb * KS:(bb + 1) * KS]
            kcat = jnp.concatenate([kb] * NH, axis=1)
            kbd_s[bb] = (kcat * kms_ref[:]).astype(bf16)
            vb = v[bb * KS:(bb + 1) * KS, :]
            vcat = jnp.concatenate([vb] * NH, axis=0)
            vbd_s[bb] = (vcat * vm_ref[:]).astype(bf16)
        wqb_s[:] = wqkv_ref[:C, :].astype(bf16)
        wpb_s[:] = wp_ref[:].astype(bf16)

    CH = TS // 2
    for c in range(2):
        rows = pl.ds(c * CH, CH)
        xb = x_ref[0, rows, :].astype(bf16)
        q = _dot(xb, wqb_s[:], ((1,), (1,)))
        lg = _dot(q.astype(bf16), kbd_s[b], ((1,), (0,)))
        e = jnp.exp(lg) * m_ref[rows, :]
        s = _dot(e, g_ref[:], ((1,), (0,)))
        r = 1.0 / jnp.maximum(s, 1e-30)
        rf = _dot(r, gt_ref[:], ((1,), (0,)))
        av = _dot((e * rf).astype(bf16), vbd_s[b], ((1,), (0,)))
        o_ref[0, rows, :] = _dot(av.astype(bf16), wpb_s[:], ((1,), (1,)))


def kernel(x, Wqkv, Wproj, Wr):
    del Wr
    f32 = jnp.float32
    bf16 = jnp.bfloat16
    x16 = x[:, :KS, :].reshape(B * KS, C)

    rows_c = jnp.arange(C)[:, None] // HD
    cols_g = jnp.arange(G)[None, :] // KS
    kms = jnp.where(rows_c == cols_g, f32(1.0 / (HD ** 0.5)), f32(0.0))
    vm = jnp.where(cols_g.T == rows_c.T, f32(1.0), f32(0.0))
    t_ids = jnp.arange(T)[:, None]
    j_ids = (jnp.arange(G) % KS)[None, :]
    mtab = jnp.where(j_ids <= t_ids, f32(1.0), f32(0.0))
    h_ids = jnp.arange(16)[None, :]
    gmat = jnp.where(cols_g.T == h_ids, f32(1.0), f32(0.0))
    gtmat = gmat.T

    out = pl.pallas_call(
        _fused_kernel,
        grid=(B, T // TS),
        in_specs=[
            pl.BlockSpec((1, TS, C), lambda b, i: (b, i, 0)),
            pl.BlockSpec((B * KS, C), lambda b, i: (0, 0)),
            pl.BlockSpec((3 * C, C), lambda b, i: (0, 0)),
            pl.BlockSpec((C, C), lambda b, i: (0, 0)),
            pl.BlockSpec((C, G), lambda b, i: (0, 0)),
            pl.BlockSpec((G, C), lambda b, i: (0, 0)),
            pl.BlockSpec((TS, G), lambda b, i: (i, 0)),
            pl.BlockSpec((G, 16), lambda b, i: (0, 0)),
            pl.BlockSpec((16, G), lambda b, i: (0, 0)),
        ],
        out_specs=pl.BlockSpec((1, TS, C), lambda b, i: (b, i, 0)),
        out_shape=jax.ShapeDtypeStruct((B, T, C), f32),
        scratch_shapes=[
            pltpu.VMEM((B, C, G), bf16),
            pltpu.VMEM((B, G, C), bf16),
            pltpu.VMEM((C, C), bf16),
            pltpu.VMEM((C, C), bf16),
        ],
        compiler_params=pltpu.CompilerParams(
            dimension_semantics=("arbitrary", "arbitrary")),
    )(x, x16, Wqkv, Wproj, kms, vm, mtab, gmat, gtmat)
    return out


# speedup vs baseline: 1.0728x; 1.0728x over previous
import jax
import jax.numpy as jnp
from jax.experimental import pallas as pl
from jax.experimental.pallas import tpu as pltpu

B, T, C, NH, KS = 4, 2048, 768, 12, 16
HD = C // NH
G = NH * KS
TS = 1024


def _dot(a, b, dims):
    return jax.lax.dot_general(a, b, (dims, ((), ())),
                               preferred_element_type=jnp.float32)


def _fused_kernel(x_ref, x16_ref, wqkv_ref, wp_ref, kms_ref, vm_ref, m_ref,
                  g_ref, gt_ref, o_ref, kbd_s, vbd_s, wqb_s, wpb_s):
    b = pl.program_id(0)
    i = pl.program_id(1)
    bf16 = jnp.bfloat16

    @pl.when((b == 0) & (i == 0))
    def _init():
        x16 = x16_ref[:].reshape(B * KS, C)
        kT = _dot(wqkv_ref[C:2 * C, :], x16, ((1,), (1,)))
        v = _dot(x16, wqkv_ref[2 * C:, :], ((1,), (1,)))
        for bb in range(B):
            kb = kT[:, bb * KS:(bb + 1) * KS]
            kcat = jnp.concatenate([kb] * NH, axis=1)
            kbd_s[bb] = (kcat * kms_ref[:]).astype(bf16)
            vb = v[bb * KS:(bb + 1) * KS, :]
            vcat = jnp.concatenate([vb] * NH, axis=0)
            vbd_s[bb] = (vcat * vm_ref[:]).astype(bf16)
        wqb_s[:] = wqkv_ref[:C, :].astype(bf16)
        wpb_s[:] = wp_ref[:].astype(bf16)

    xb = x_ref[0].astype(bf16)
    q = _dot(xb, wqb_s[:], ((1,), (1,)))
    lg = _dot(q.astype(bf16), kbd_s[b], ((1,), (0,)))
    e = jnp.exp(lg) * m_ref[:]
    s = _dot(e, g_ref[:], ((1,), (0,)))
    r = 1.0 / jnp.maximum(s, 1e-30)
    rf = _dot(r, gt_ref[:], ((1,), (0,)))
    av = _dot((e * rf).astype(bf16), vbd_s[b], ((1,), (0,)))
    o_ref[0] = _dot(av.astype(bf16), wpb_s[:], ((1,), (1,)))


def kernel(x, Wqkv, Wproj, Wr):
    del Wr
    f32 = jnp.float32
    bf16 = jnp.bfloat16

    rows_c = jnp.arange(C)[:, None] // HD
    cols_g = jnp.arange(G)[None, :] // KS
    kms = jnp.where(rows_c == cols_g, f32(1.0 / (HD ** 0.5)), f32(0.0))
    vm = jnp.where(cols_g.T == rows_c.T, f32(1.0), f32(0.0))
    t_ids = jnp.arange(T)[:, None]
    j_ids = (jnp.arange(G) % KS)[None, :]
    mtab = jnp.where(j_ids <= t_ids, f32(1.0), f32(0.0))
    h_ids = jnp.arange(16)[None, :]
    gmat = jnp.where(cols_g.T == h_ids, f32(1.0), f32(0.0))
    gtmat = gmat.T

    out = pl.pallas_call(
        _fused_kernel,
        grid=(B, T // TS),
        in_specs=[
            pl.BlockSpec((1, TS, C), lambda b, i: (b, i, 0)),
            pl.BlockSpec((B, KS, C), lambda b, i: (0, 0, 0)),
            pl.BlockSpec((3 * C, C), lambda b, i: (0, 0)),
            pl.BlockSpec((C, C), lambda b, i: (0, 0)),
            pl.BlockSpec((C, G), lambda b, i: (0, 0)),
            pl.BlockSpec((G, C), lambda b, i: (0, 0)),
            pl.BlockSpec((TS, G), lambda b, i: (i, 0)),
            pl.BlockSpec((G, 16), lambda b, i: (0, 0)),
            pl.BlockSpec((16, G), lambda b, i: (0, 0)),
        ],
        out_specs=pl.BlockSpec((1, TS, C), lambda b, i: (b, i, 0)),
        out_shape=jax.ShapeDtypeStruct((B, T, C), f32),
        scratch_shapes=[
            pltpu.VMEM((B, C, G), bf16),
            pltpu.VMEM((B, G, C), bf16),
            pltpu.VMEM((C, C), bf16),
            pltpu.VMEM((C, C), bf16),
        ],
        compiler_params=pltpu.CompilerParams(
            dimension_semantics=("arbitrary", "arbitrary")),
    )(x, x, Wqkv, Wproj, kms, vm, mtab, gmat, gtmat)
    return out


# speedup vs baseline: 1.4906x; 1.3894x over previous
import jax
import jax.numpy as jnp
from jax.experimental import pallas as pl
from jax.experimental.pallas import tpu as pltpu

B, T, C, NH, KS = 4, 2048, 768, 12, 16
HD = C // NH
G = NH * KS
TS = 1024


def _dot(a, b, dims):
    return jax.lax.dot_general(a, b, (dims, ((), ())),
                               preferred_element_type=jnp.float32)


def _fused_kernel(x_ref, x16_ref, wqkv_ref, wp_ref, kms_ref, vm_ref, m_ref,
                  g_ref, gt_ref, o_ref, m_s, p_s):
    b = pl.program_id(0)
    i = pl.program_id(1)
    bf16 = jnp.bfloat16

    @pl.when((b == 0) & (i == 0))
    def _init():
        x16 = x16_ref[:].reshape(B * KS, C)
        kT = _dot(wqkv_ref[C:2 * C, :], x16, ((1,), (1,)))
        v = _dot(x16, wqkv_ref[2 * C:, :], ((1,), (1,)))
        wqb = wqkv_ref[:C, :].astype(bf16)
        wpb = wp_ref[:].astype(bf16)
        for bb in range(B):
            kb = kT[:, bb * KS:(bb + 1) * KS]
            kcat = jnp.concatenate([kb] * NH, axis=1)
            kbd = (kcat * kms_ref[:]).astype(bf16)
            m_s[bb] = _dot(wqb, kbd, ((0,), (0,))).astype(bf16)
            vb = v[bb * KS:(bb + 1) * KS, :]
            vcat = jnp.concatenate([vb] * NH, axis=0)
            vbd = (vcat * vm_ref[:]).astype(bf16)
            p_s[bb] = _dot(vbd, wpb, ((1,), (1,))).astype(bf16)

    xb = x_ref[0].astype(bf16)
    lg = _dot(xb, m_s[b], ((1,), (0,)))
    e = jnp.exp(lg) * m_ref[:]
    s = _dot(e, g_ref[:], ((1,), (0,)))
    r = 1.0 / jnp.maximum(s, 1e-30)
    rf = _dot(r, gt_ref[:], ((1,), (0,)))
    o_ref[0] = _dot((e * rf).astype(bf16), p_s[b], ((1,), (0,)))


def kernel(x, Wqkv, Wproj, Wr):
    del Wr
    f32 = jnp.float32
    bf16 = jnp.bfloat16

    rows_c = jnp.arange(C)[:, None] // HD
    cols_g = jnp.arange(G)[None, :] // KS
    kms = jnp.where(rows_c == cols_g, f32(1.0 / (HD ** 0.5)), f32(0.0))
    vm = jnp.where(cols_g.T == rows_c.T, f32(1.0), f32(0.0))
    t_ids = jnp.arange(T)[:, None]
    j_ids = (jnp.arange(G) % KS)[None, :]
    mtab = jnp.where(j_ids <= t_ids, f32(1.0), f32(0.0))
    h_ids = jnp.arange(16)[None, :]
    gmat = jnp.where(cols_g.T == h_ids, f32(1.0), f32(0.0))
    gtmat = gmat.T

    out = pl.pallas_call(
        _fused_kernel,
        grid=(B, T // TS),
        in_specs=[
            pl.BlockSpec((1, TS, C), lambda b, i: (b, i, 0)),
            pl.BlockSpec((B, KS, C), lambda b, i: (0, 0, 0)),
            pl.BlockSpec((3 * C, C), lambda b, i: (0, 0)),
            pl.BlockSpec((C, C), lambda b, i: (0, 0)),
            pl.BlockSpec((C, G), lambda b, i: (0, 0)),
            pl.BlockSpec((G, C), lambda b, i: (0, 0)),
            pl.BlockSpec((TS, G), lambda b, i: (i, 0)),
            pl.BlockSpec((G, 16), lambda b, i: (0, 0)),
            pl.BlockSpec((16, G), lambda b, i: (0, 0)),
        ],
        out_specs=pl.BlockSpec((1, TS, C), lambda b, i: (b, i, 0)),
        out_shape=jax.ShapeDtypeStruct((B, T, C), f32),
        scratch_shapes=[
            pltpu.VMEM((B, C, G), bf16),
            pltpu.VMEM((B, G, C), bf16),
        ],
        compiler_params=pltpu.CompilerParams(
            dimension_semantics=("arbitrary", "arbitrary")),
    )(x, x, Wqkv, Wproj, kms, vm, mtab, gmat, gtmat)
    return out


# speedup vs baseline: 1.5851x; 1.0634x over previous
import jax
import jax.numpy as jnp
from jax.experimental import pallas as pl
from jax.experimental.pallas import tpu as pltpu

B, T, C, NH, KS = 4, 2048, 768, 12, 16
HD = C // NH
G = NH * KS
TS = 1024


def _dot(a, b, dims):
    return jax.lax.dot_general(a, b, (dims, ((), ())),
                               preferred_element_type=jnp.float32)


def _fused_kernel(x_ref, x16_ref, wqkv_ref, wp_ref, kms_ref, vm_ref, m_ref,
                  g_ref, gt_ref, o_ref, m_s, p_s):
    b = pl.program_id(0)
    i = pl.program_id(1)
    bf16 = jnp.bfloat16

    @pl.when((b == 0) & (i == 0))
    def _init():
        x16 = x16_ref[:].reshape(B * KS, C)
        kT = _dot(wqkv_ref[C:2 * C, :], x16, ((1,), (1,)))
        v = _dot(x16, wqkv_ref[2 * C:, :], ((1,), (1,)))
        wqb = wqkv_ref[:C, :].astype(bf16)
        wpb = wp_ref[:].astype(bf16)
        for bb in range(B):
            kb = kT[:, bb * KS:(bb + 1) * KS]
            kcat = jnp.concatenate([kb] * NH, axis=1)
            kbd = (kcat * kms_ref[:]).astype(bf16)
            m_s[bb] = _dot(wqb, kbd, ((0,), (0,))).astype(bf16)
            vb = v[bb * KS:(bb + 1) * KS, :]
            vcat = jnp.concatenate([vb] * NH, axis=0)
            vbd = (vcat * vm_ref[:]).astype(bf16)
            p_s[bb] = _dot(vbd, wpb, ((1,), (1,))).astype(bf16)

    xb = x_ref[0].astype(bf16)
    lg = _dot(xb, m_s[b], ((1,), (0,)))
    e = jnp.exp(lg)
    e = jnp.concatenate([e[:KS] * m_ref[:], e[KS:]], axis=0)
    s = _dot(e, g_ref[:], ((1,), (0,)))
    r = 1.0 / jnp.maximum(s, 1e-30)
    rf = _dot(r, gt_ref[:], ((1,), (0,)))
    o_ref[0] = _dot((e * rf).astype(bf16), p_s[b], ((1,), (0,)))


def kernel(x, Wqkv, Wproj, Wr):
    del Wr
    f32 = jnp.float32
    bf16 = jnp.bfloat16

    rows_c = jnp.arange(C)[:, None] // HD
    cols_g = jnp.arange(G)[None, :] // KS
    kms = jnp.where(rows_c == cols_g, f32(1.0 / (HD ** 0.5)), f32(0.0))
    vm = jnp.where(cols_g.T == rows_c.T, f32(1.0), f32(0.0))
    t_ids = jnp.arange(KS)[:, None]
    j_ids = (jnp.arange(G) % KS)[None, :]
    mtab = jnp.concatenate(
        [jnp.where(j_ids <= t_ids, f32(1.0), f32(0.0)),
         jnp.ones((KS, G), f32)], axis=0)
    h_ids = jnp.arange(16)[None, :]
    gmat = jnp.where(cols_g.T == h_ids, f32(1.0), f32(0.0))
    gtmat = gmat.T

    out = pl.pallas_call(
        _fused_kernel,
        grid=(B, T // TS),
        in_specs=[
            pl.BlockSpec((1, TS, C), lambda b, i: (b, i, 0)),
            pl.BlockSpec((B, KS, C), lambda b, i: (0, 0, 0)),
            pl.BlockSpec((3 * C, C), lambda b, i: (0, 0)),
            pl.BlockSpec((C, C), lambda b, i: (0, 0)),
            pl.BlockSpec((C, G), lambda b, i: (0, 0)),
            pl.BlockSpec((G, C), lambda b, i: (0, 0)),
            pl.BlockSpec((KS, G), lambda b, i: (jnp.minimum(i, 1), 0)),
            pl.BlockSpec((G, 16), lambda b, i: (0, 0)),
            pl.BlockSpec((16, G), lambda b, i: (0, 0)),
        ],
        out_specs=pl.BlockSpec((1, TS, C), lambda b, i: (b, i, 0)),
        out_shape=jax.ShapeDtypeStruct((B, T, C), f32),
        scratch_shapes=[
            pltpu.VMEM((B, C, G), bf16),
            pltpu.VMEM((B, G, C), bf16),
        ],
        compiler_params=pltpu.CompilerParams(
            dimension_semantics=("arbitrary", "arbitrary")),
    )(x, x, Wqkv, Wproj, kms, vm, mtab, gmat, gtmat)
    return out


# speedup vs baseline: 1.6554x; 1.0443x over previous
import jax
import jax.numpy as jnp
from jax.experimental import pallas as pl
from jax.experimental.pallas import tpu as pltpu

B, T, C, NH, KS = 4, 2048, 768, 12, 16
HD = C // NH
G = NH * KS
TS = 2048


def _dot(a, b, dims):
    return jax.lax.dot_general(a, b, (dims, ((), ())),
                               preferred_element_type=jnp.float32)


def _fused_kernel(x_ref, x16_ref, wqkv_ref, wp_ref, kms_ref, vm_ref, m_ref,
                  g_ref, gt_ref, o_ref, m_s, p_s):
    b = pl.program_id(0)
    i = pl.program_id(1)
    bf16 = jnp.bfloat16

    @pl.when((b == 0) & (i == 0))
    def _init():
        x16 = x16_ref[:].reshape(B * KS, C)
        kT = _dot(wqkv_ref[C:2 * C, :], x16, ((1,), (1,)))
        v = _dot(x16, wqkv_ref[2 * C:, :], ((1,), (1,)))
        wqb = wqkv_ref[:C, :].astype(bf16)
        wpb = wp_ref[:].astype(bf16)
        for bb in range(B):
            kb = kT[:, bb * KS:(bb + 1) * KS]
            kcat = jnp.concatenate([kb] * NH, axis=1)
            kbd = (kcat * kms_ref[:]).astype(bf16)
            m_s[bb] = _dot(wqb, kbd, ((0,), (0,))).astype(bf16)
            vb = v[bb * KS:(bb + 1) * KS, :]
            vcat = jnp.concatenate([vb] * NH, axis=0)
            vbd = (vcat * vm_ref[:]).astype(bf16)
            p_s[bb] = _dot(vbd, wpb, ((1,), (1,))).astype(bf16)

    xb = x_ref[0].astype(bf16)
    lg = _dot(xb, m_s[b], ((1,), (0,)))
    e = jnp.exp(lg)
    e = jnp.concatenate([e[:KS] * m_ref[:], e[KS:]], axis=0)
    s = _dot(e, g_ref[:], ((1,), (0,)))
    r = 1.0 / jnp.maximum(s, 1e-30)
    rf = _dot(r, gt_ref[:], ((1,), (0,)))
    o_ref[0] = _dot((e * rf).astype(bf16), p_s[b], ((1,), (0,)))


def kernel(x, Wqkv, Wproj, Wr):
    del Wr
    f32 = jnp.float32
    bf16 = jnp.bfloat16

    rows_c = jnp.arange(C)[:, None] // HD
    cols_g = jnp.arange(G)[None, :] // KS
    kms = jnp.where(rows_c == cols_g, f32(1.0 / (HD ** 0.5)), f32(0.0))
    vm = jnp.where(cols_g.T == rows_c.T, f32(1.0), f32(0.0))
    t_ids = jnp.arange(KS)[:, None]
    j_ids = (jnp.arange(G) % KS)[None, :]
    mtab = jnp.concatenate(
        [jnp.where(j_ids <= t_ids, f32(1.0), f32(0.0)),
         jnp.ones((KS, G), f32)], axis=0)
    h_ids = jnp.arange(16)[None, :]
    gmat = jnp.where(cols_g.T == h_ids, f32(1.0), f32(0.0))
    gtmat = gmat.T

    out = pl.pallas_call(
        _fused_kernel,
        grid=(B, T // TS),
        in_specs=[
            pl.BlockSpec((1, TS, C), lambda b, i: (b, i, 0)),
            pl.BlockSpec((B, KS, C), lambda b, i: (0, 0, 0)),
            pl.BlockSpec((3 * C, C), lambda b, i: (0, 0)),
            pl.BlockSpec((C, C), lambda b, i: (0, 0)),
            pl.BlockSpec((C, G), lambda b, i: (0, 0)),
            pl.BlockSpec((G, C), lambda b, i: (0, 0)),
            pl.BlockSpec((KS, G), lambda b, i: (jnp.minimum(i, 1), 0)),
            pl.BlockSpec((G, 16), lambda b, i: (0, 0)),
            pl.BlockSpec((16, G), lambda b, i: (0, 0)),
        ],
        out_specs=pl.BlockSpec((1, TS, C), lambda b, i: (b, i, 0)),
        out_shape=jax.ShapeDtypeStruct((B, T, C), f32),
        scratch_shapes=[
            pltpu.VMEM((B, C, G), bf16),
            pltpu.VMEM((B, G, C), bf16),
        ],
        compiler_params=pltpu.CompilerParams(
            dimension_semantics=("arbitrary", "arbitrary")),
    )(x, x, Wqkv, Wproj, kms, vm, mtab, gmat, gtmat)
    return out


# speedup vs baseline: 1.6568x; 1.0009x over previous
import jax
import jax.numpy as jnp
from jax.experimental import pallas as pl
from jax.experimental.pallas import tpu as pltpu

B, T, C, NH, KS = 4, 2048, 768, 12, 16
HD = C // NH
G = NH * KS


def _dot(a, b, dims):
    return jax.lax.dot_general(a, b, (dims, ((), ())),
                               preferred_element_type=jnp.float32)


def _fused_kernel(x_ref, x16_ref, wqkv_ref, wp_ref, kms_ref, vm_ref, m_ref,
                  g_ref, gt_ref, o_ref):
    bf16 = jnp.bfloat16
    x16 = x16_ref[0]
    kT = _dot(wqkv_ref[C:2 * C, :], x16, ((1,), (1,)))
    v16 = _dot(x16, wqkv_ref[2 * C:, :], ((1,), (1,)))
    kbd = (jnp.concatenate([kT] * NH, axis=1) * kms_ref[:]).astype(bf16)
    vbd = (jnp.concatenate([v16] * NH, axis=0) * vm_ref[:]).astype(bf16)
    m_mat = _dot(wqkv_ref[:C, :].astype(bf16), kbd, ((0,), (0,))).astype(bf16)
    p_mat = _dot(vbd, wp_ref[:].astype(bf16), ((1,), (1,))).astype(bf16)

    xb = x_ref[0].astype(bf16)
    lg = _dot(xb, m_mat, ((1,), (0,)))
    e = jnp.exp(lg)
    e = jnp.concatenate([e[:KS] * m_ref[:], e[KS:]], axis=0)
    s = _dot(e, g_ref[:], ((1,), (0,)))
    r = 1.0 / jnp.maximum(s, 1e-30)
    rf = _dot(r, gt_ref[:], ((1,), (0,)))
    o_ref[0] = _dot((e * rf).astype(bf16), p_mat, ((1,), (0,)))


def kernel(x, Wqkv, Wproj, Wr):
    del Wr
    f32 = jnp.float32

    rows_c = jnp.arange(C)[:, None] // HD
    cols_g = jnp.arange(G)[None, :] // KS
    kms = jnp.where(rows_c == cols_g, f32(1.0 / (HD ** 0.5)), f32(0.0))
    vm = jnp.where(cols_g.T == rows_c.T, f32(1.0), f32(0.0))
    t_ids = jnp.arange(KS)[:, None]
    j_ids = (jnp.arange(G) % KS)[None, :]
    mtab = jnp.where(j_ids <= t_ids, f32(1.0), f32(0.0))
    h_ids = jnp.arange(16)[None, :]
    gmat = jnp.where(cols_g.T == h_ids, f32(1.0), f32(0.0))
    gtmat = gmat.T

    out = pl.pallas_call(
        _fused_kernel,
        grid=(B,),
        in_specs=[
            pl.BlockSpec((1, T, C), lambda b: (b, 0, 0)),
            pl.BlockSpec((1, KS, C), lambda b: (b, 0, 0)),
            pl.BlockSpec((3 * C, C), lambda b: (0, 0)),
            pl.BlockSpec((C, C), lambda b: (0, 0)),
            pl.BlockSpec((C, G), lambda b: (0, 0)),
            pl.BlockSpec((G, C), lambda b: (0, 0)),
            pl.BlockSpec((KS, G), lambda b: (0, 0)),
            pl.BlockSpec((G, 16), lambda b: (0, 0)),
            pl.BlockSpec((16, G), lambda b: (0, 0)),
        ],
        out_specs=pl.BlockSpec((1, T, C), lambda b: (b, 0, 0)),
        out_shape=jax.ShapeDtypeStruct((B, T, C), f32),
        compiler_params=pltpu.CompilerParams(
            dimension_semantics=("arbitrary",)),
    )(x, x, Wqkv, Wproj, kms, vm, mtab, gmat, gtmat)
    return out
